# baseline (speedup 1.0000x reference)
import jax
import jax.numpy as jnp
from jax.experimental import pallas as pl

D = 128
AW = 144
N_NODES = 50000


def _w_kernel(ea_ref, We_ref, be_ref, out_ref):
    x = ea_ref[0]
    y = jax.nn.relu(
        jnp.dot(x, We_ref[...], preferred_element_type=jnp.float32) + be_ref[...]
    )
    out_ref[0, 0, :] = jnp.mean(y, axis=1)


def _edge_w(ea, We, be):
    E, ed = ea.shape
    B = 1000
    nb = E // B
    ea_p = jnp.zeros((E, 16), jnp.float32).at[:, :ed].set(ea).reshape(nb, B, 16)
    We_p = jnp.zeros((16, D), jnp.float32).at[:ed, :].set(We)
    out = pl.pallas_call(
        _w_kernel,
        grid=(nb,),
        in_specs=[
            pl.BlockSpec((1, B, 16), lambda i: (i, 0, 0)),
            pl.BlockSpec((16, D), lambda i: (0, 0)),
            pl.BlockSpec((1, D), lambda i: (0, 0)),
        ],
        out_specs=pl.BlockSpec((1, 1, B), lambda i: (i, 0, 0)),
        out_shape=jax.ShapeDtypeStruct((nb, 1, B), jnp.float32),
    )(ea_p, We_p, be.reshape(1, D))
    return out.reshape(E)


def _combine2_kernel(acc1_ref, acc2_ref, h_ref, Wl1_ref, Wl2_ref, Wr_ref,
                     bias_ref, g_ref, b_ref, out_ref, *, residual):
    a1 = acc1_ref[:, :D]
    c1 = acc1_ref[:, D:D + 1]
    a2 = acc2_ref[:, :D]
    c2 = acc2_ref[:, D:D + 1]
    h = h_ref[...]
    y = jnp.dot(a1 / jnp.maximum(c1, 1.0), Wl1_ref[...],
                preferred_element_type=jnp.float32)
    y = y + jnp.dot(a2 / jnp.maximum(c2, 1.0), Wl2_ref[...],
                    preferred_element_type=jnp.float32)
    y = y + jnp.dot(h, Wr_ref[...], preferred_element_type=jnp.float32)
    y = y + bias_ref[...]
    y = jax.nn.relu(y)
    mu = jnp.mean(y, axis=1, keepdims=True)
    yc = y - mu
    var = jnp.mean(yc * yc, axis=1, keepdims=True)
    out = yc * jax.lax.rsqrt(var + 1e-5) * g_ref[...] + b_ref[...]
    if residual:
        out = out + h
    out_ref[...] = out


def _combine1_kernel(acc1_ref, h_ref, Wl1_ref, Wr_ref,
                     bias_ref, g_ref, b_ref, out_ref, *, residual):
    a1 = acc1_ref[:, :D]
    c1 = acc1_ref[:, D:D + 1]
    h = h_ref[...]
    y = jnp.dot(a1 / jnp.maximum(c1, 1.0), Wl1_ref[...],
                preferred_element_type=jnp.float32)
    y = y + jnp.dot(h, Wr_ref[...], preferred_element_type=jnp.float32)
    y = y + bias_ref[...]
    y = jax.nn.relu(y)
    mu = jnp.mean(y, axis=1, keepdims=True)
    yc = y - mu
    var = jnp.mean(yc * yc, axis=1, keepdims=True)
    out = yc * jax.lax.rsqrt(var + 1e-5) * g_ref[...] + b_ref[...]
    if residual:
        out = out + h
    out_ref[...] = out


def _combine2(acc1, acc2, h, Wl1, Wl2, Wr, bias, g, b, residual):
    import functools
    N = h.shape[0]
    B = 1000
    nb = N // B
    body = functools.partial(_combine2_kernel, residual=residual)
    return pl.pallas_call(
        body,
        grid=(nb,),
        in_specs=[
            pl.BlockSpec((B, AW), lambda i: (i, 0)),
            pl.BlockSpec((B, AW), lambda i: (i, 0)),
            pl.BlockSpec((B, D), lambda i: (i, 0)),
            pl.BlockSpec((D, D), lambda i: (0, 0)),
            pl.BlockSpec((D, D), lambda i: (0, 0)),
            pl.BlockSpec((D, D), lambda i: (0, 0)),
            pl.BlockSpec((1, D), lambda i: (0, 0)),
            pl.BlockSpec((1, D), lambda i: (0, 0)),
            pl.BlockSpec((1, D), lambda i: (0, 0)),
        ],
        out_specs=pl.BlockSpec((B, D), lambda i: (i, 0)),
        out_shape=jax.ShapeDtypeStruct((N, D), jnp.float32),
    )(acc1, acc2, h, Wl1, Wl2, Wr, bias.reshape(1, D), g.reshape(1, D),
      b.reshape(1, D))


def _combine1(acc1, h, Wl1, Wr, bias, g, b, residual):
    import functools
    N = h.shape[0]
    B = 1000
    nb = N // B
    body = functools.partial(_combine1_kernel, residual=residual)
    return pl.pallas_call(
        body,
        grid=(nb,),
        in_specs=[
            pl.BlockSpec((B, AW), lambda i: (i, 0)),
            pl.BlockSpec((B, D), lambda i: (i, 0)),
            pl.BlockSpec((D, D), lambda i: (0, 0)),
            pl.BlockSpec((D, D), lambda i: (0, 0)),
            pl.BlockSpec((1, D), lambda i: (0, 0)),
            pl.BlockSpec((1, D), lambda i: (0, 0)),
            pl.BlockSpec((1, D), lambda i: (0, 0)),
        ],
        out_specs=pl.BlockSpec((B, D), lambda i: (i, 0)),
        out_shape=jax.ShapeDtypeStruct((N, D), jnp.float32),
    )(acc1, h, Wl1, Wr, bias.reshape(1, D), g.reshape(1, D), b.reshape(1, D))


def _agg_jax(x_src, src, dst, w, num_dst):
    msg = x_src[src] * w[:, None]
    s = jax.ops.segment_sum(msg, dst, num_segments=num_dst)
    c = jax.ops.segment_sum(jnp.ones((src.shape[0],), jnp.float32), dst,
                            num_segments=num_dst)
    return jnp.concatenate(
        [s, c[:, None], jnp.zeros((num_dst, AW - D - 1), jnp.float32)], axis=1)


def kernel(x_user, x_place, ei_uu, ea_uu, ei_up, ea_up, ei_pu, ea_pu, params):
    h_u, h_p = x_user, x_place
    for l in range(2):
        lp = params['layer%d' % l]
        w_uu = _edge_w(ea_uu, lp['uu']['We'], lp['uu']['be'])
        w_pu = _edge_w(ea_pu, lp['pu']['We'], lp['pu']['be'])
        w_up = _edge_w(ea_up, lp['up']['We'], lp['up']['be'])
        acc_uu = _agg_jax(h_u, ei_uu[0], ei_uu[1], w_uu, N_NODES)
        acc_pu = _agg_jax(h_p, ei_pu[0], ei_pu[1], w_pu, N_NODES)
        acc_up = _agg_jax(h_u, ei_up[0], ei_up[1], w_up, N_NODES)
        bias_u = (lp['uu']['bl'] + lp['uu']['br'] + lp['pu']['bl'] +
                  lp['pu']['br'])
        bias_p = lp['up']['bl'] + lp['up']['br']
        new_u = _combine2(acc_uu, acc_pu, h_u, lp['uu']['Wl'], lp['pu']['Wl'],
                          lp['uu']['Wr'] + lp['pu']['Wr'], bias_u,
                          lp['ln_u_g'], lp['ln_u_b'], residual=(l > 0))
        new_p = _combine1(acc_up, h_p, lp['up']['Wl'], lp['up']['Wr'], bias_p,
                          lp['ln_p_g'], lp['ln_p_b'], residual=(l > 0))
        h_u, h_p = new_u, new_p
    return h_u, h_p
